# baseline (speedup 1.0000x reference)
import functools

import jax
import jax.numpy as jnp
from jax import lax
from jax.experimental import pallas as pl
from jax.experimental.pallas import tpu as pltpu
from jax.experimental.pallas import tpu_sc as plsc

D = 32
CHUNK = 256
NBUF = 4
NW = 32


@functools.lru_cache(maxsize=None)
def _make_kernel(B):
    b_per_w = B // NW
    nch = b_per_w // CHUNK
    assert nch % NBUF == 0
    mesh = plsc.VectorSubcoreMesh(core_axis_name="c", subcore_axis_name="s")

    @functools.partial(
        pl.kernel,
        mesh=mesh,
        compiler_params=pltpu.CompilerParams(use_tc_tiling_on_sc=False),
        out_type=jax.ShapeDtypeStruct((B, D), jnp.float32),
        scratch_types=[
            pltpu.VMEM((nch, CHUNK), jnp.int32),
            *([pltpu.VMEM((CHUNK, D), jnp.float32)] * NBUF),
            *([pltpu.SemaphoreType.DMA] * NBUF),
            *([pltpu.SemaphoreType.DMA] * NBUF),
        ],
    )
    def k(x_hbm, w_hbm, out_hbm, idx_v, *bufs_and_sems):
        rows = bufs_and_sems[:NBUF]
        gsem = bufs_and_sems[NBUF:2 * NBUF]
        ssem = bufs_and_sems[2 * NBUF:]
        c = lax.axis_index("c")
        s = lax.axis_index("s")
        wid = s * 2 + c
        base = wid * b_per_w

        pltpu.sync_copy(x_hbm.at[wid], idx_v)

        for b in range(NBUF):
            pltpu.async_copy(w_hbm.at[idx_v.at[b]], rows[b], gsem[b])

        def body(i, carry):
            for b in range(NBUF):
                j = i * NBUF + b
                bp = (b - 1) % NBUF
                pltpu.make_async_copy(
                    w_hbm.at[idx_v.at[j]], rows[b], gsem[b]
                ).wait()
                out_ref = out_hbm.at[pl.ds(base + j * CHUNK, CHUNK)]
                pltpu.async_copy(rows[b], out_ref, ssem[b])

                @pl.when((j >= 1) & (j - 1 + NBUF < nch))
                def _():
                    jp = j - 1
                    prev_out = out_hbm.at[pl.ds(base + jp * CHUNK, CHUNK)]
                    pltpu.make_async_copy(rows[bp], prev_out, ssem[bp]).wait()
                    pltpu.async_copy(
                        w_hbm.at[idx_v.at[jp + NBUF]], rows[bp], gsem[bp]
                    )

            return carry

        lax.fori_loop(0, nch // NBUF, body, 0)

        for b in range(NBUF):
            jt = nch - NBUF + b
            pltpu.make_async_copy(
                rows[b],
                out_hbm.at[pl.ds(base + jt * CHUNK, CHUNK)],
                ssem[b],
            ).wait()

    return k


def kernel(x, weight):
    BATCH, FIELDS = x.shape
    B = BATCH * FIELDS
    x_flat = x.reshape(NW, (B // NW) // CHUNK, CHUNK).astype(jnp.int32)
    out = _make_kernel(B)(x_flat, weight)
    return out.reshape(BATCH, FIELDS, D)
